# Initial kernel scaffold; baseline (speedup 1.0000x reference)
import jax
import jax.numpy as jnp
from jax.experimental import pallas as pl


def kernel(X, params):
    raise NotImplementedError("write your pallas kernel here")


# speedup vs baseline: 9.5011x; 9.5011x over previous
import functools

import jax
import jax.numpy as jnp
from jax import lax
from jax.experimental import pallas as pl
from jax.experimental.pallas import tpu as pltpu
from jax.experimental.pallas import tpu_sc as plsc

B = 2048
G = 128
ARITY = 3
OUT_DIM = 64
NTILES = 32
GPT = G // NTILES
NB = 256
NPAIRS = NB * GPT
TBL = GPT * 27 * OUT_DIM
L = 16


def _sc_body(x_hbm, p_hbm, out_hbm, xv, tbl, outv, sem):
    wid = lax.axis_index("s") * 2 + lax.axis_index("c")

    pltpu.sync_copy(p_hbm.at[wid], tbl)

    iota = lax.iota(jnp.int32, L)
    g_loc = lax.rem(iota, GPT)

    def chunk_body(chunk, _):
        pltpu.sync_copy(x_hbm.at[wid, pl.ds(chunk * NB * 12, NB * 12)], xv)

        def blk_body(blk, _):
            base = blk * L
            p = base + iota
            ip = p * ARITY
            a0 = plsc.load_gather(xv, [ip])
            a1 = plsc.load_gather(xv, [ip + 1])
            a2 = plsc.load_gather(xv, [ip + 2])
            b0, b1, b2 = jnp.abs(a0), jnp.abs(a1), jnp.abs(a2)
            mn = jnp.minimum(jnp.minimum(b0, b1), b2)
            mx = jnp.maximum(jnp.maximum(b0, b1), b2)
            md = jnp.maximum(jnp.minimum(b0, b1),
                             jnp.minimum(jnp.maximum(b0, b1), b2))
            one = jnp.int32(1)
            t0 = jnp.where(a0 >= 0, one, -one)
            t1 = jnp.where(a1 >= 0, jnp.int32(3), jnp.int32(-3))
            t2 = jnp.where(a2 >= 0, jnp.int32(9), jnp.int32(-9))
            s_all = t0 + t1 + t2
            m1 = b1 < b0
            tmin = jnp.where(m1, t1, t0)
            bmin = jnp.where(m1, b1, b0)
            tmin = jnp.where(b2 < bmin, t2, tmin)
            m2 = b1 >= b0
            tmax = jnp.where(m2, t1, t0)
            bmax = jnp.where(m2, b1, b0)
            tmax = jnp.where(b2 >= bmax, t2, tmax)

            gbase = g_loc * 27 + 13
            r0 = (gbase + s_all) * OUT_DIM
            r1 = (gbase + (s_all - tmin)) * OUT_DIM
            r2 = (gbase + tmax) * OUT_DIM
            c0 = mn
            c1 = md - mn
            c2 = mx - md

            row = lax.div(p, GPT)
            col0 = g_loc * OUT_DIM
            for d in range(OUT_DIM):
                v0 = plsc.load_gather(tbl, [r0 + d])
                v1 = plsc.load_gather(tbl, [r1 + d])
                v2 = plsc.load_gather(tbl, [r2 + d])
                acc = c0 * v0 + c1 * v1 + c2 * v2
                plsc.store_scatter(outv, [row, col0 + d], acc)
            return _

        lax.fori_loop(0, NPAIRS // L, blk_body, None)
        pltpu.sync_copy(
            outv,
            out_hbm.at[pl.ds(chunk * NB, NB), pl.ds(wid * GPT * OUT_DIM,
                                                    GPT * OUT_DIM)])
        return _

    lax.fori_loop(0, B // NB, chunk_body, None)


@jax.jit
def _sc_call(x_t, p_t):
    mesh = plsc.VectorSubcoreMesh(core_axis_name="c", subcore_axis_name="s")
    k = pl.kernel(
        _sc_body,
        mesh=mesh,
        compiler_params=pltpu.CompilerParams(needs_layout_passes=False),
        out_type=jax.ShapeDtypeStruct((B, G * OUT_DIM), jnp.float32),
        scratch_types=[
            pltpu.VMEM((NB * 12,), jnp.float32),
            pltpu.VMEM((TBL,), jnp.float32),
            pltpu.VMEM((NB, GPT * OUT_DIM), jnp.float32),
            pltpu.SemaphoreType.DMA,
        ],
    )
    return k(x_t, p_t)


def kernel(X, params):
    x_t = X.reshape(B, NTILES, GPT * ARITY).transpose(1, 0, 2)
    x_t = x_t.reshape(NTILES, B * GPT * ARITY)
    p_t = params.reshape(NTILES, TBL)
    return _sc_call(x_t, p_t)


# speedup vs baseline: 30.7941x; 3.2411x over previous
import functools

import jax
import jax.numpy as jnp
from jax import lax
from jax.experimental import pallas as pl
from jax.experimental.pallas import tpu as pltpu
from jax.experimental.pallas import tpu_sc as plsc

B = 2048
G = 128
ARITY = 3
OUT_DIM = 64
NTILES = 32
GPT = G // NTILES
NB = 256
NPAIRS = NB * GPT
PAD = OUT_DIM + 1
TBL = GPT * 27 * PAD
L = 16


def _sc_body(x_hbm, p_hbm, out_hbm, xv, tbl, outv, sem):
    wid = lax.axis_index("s") * 2 + lax.axis_index("c")

    pltpu.sync_copy(p_hbm.at[wid], tbl)

    iota = lax.iota(jnp.int32, L)
    g_loc = iota & (GPT - 1)

    def chunk_body(chunk, _):
        pltpu.sync_copy(x_hbm.at[wid, pl.ds(chunk * NB * 12, NB * 12)], xv)

        def blk_body(blk, _):
            base = blk * L
            p = base + iota
            ip = p * ARITY
            a0 = plsc.load_gather(xv, [ip])
            a1 = plsc.load_gather(xv, [ip + 1])
            a2 = plsc.load_gather(xv, [ip + 2])
            b0, b1, b2 = jnp.abs(a0), jnp.abs(a1), jnp.abs(a2)
            mn = jnp.minimum(jnp.minimum(b0, b1), b2)
            mx = jnp.maximum(jnp.maximum(b0, b1), b2)
            md = jnp.maximum(jnp.minimum(b0, b1),
                             jnp.minimum(jnp.maximum(b0, b1), b2))
            one = jnp.int32(1)
            t0 = jnp.where(a0 >= 0, one, -one)
            t1 = jnp.where(a1 >= 0, jnp.int32(3), jnp.int32(-3))
            t2 = jnp.where(a2 >= 0, jnp.int32(9), jnp.int32(-9))
            s_all = t0 + t1 + t2
            m1 = b1 < b0
            tmin = jnp.where(m1, t1, t0)
            bmin = jnp.where(m1, b1, b0)
            tmin = jnp.where(b2 < bmin, t2, tmin)
            m2 = b1 >= b0
            tmax = jnp.where(m2, t1, t0)
            bmax = jnp.where(m2, b1, b0)
            tmax = jnp.where(b2 >= bmax, t2, tmax)

            gbase = g_loc * 27 + 13
            r0 = (gbase + s_all) * PAD
            r1 = (gbase + (s_all - tmin)) * PAD
            r2 = (gbase + tmax) * PAD
            c0 = mn
            c1 = md - mn
            c2 = mx - md

            row = lax.shift_right_logical(p, 2)
            col0 = g_loc * OUT_DIM
            for d in range(OUT_DIM):
                dl = (iota + d) & (OUT_DIM - 1)
                v0 = plsc.load_gather(tbl, [r0 + dl])
                v1 = plsc.load_gather(tbl, [r1 + dl])
                v2 = plsc.load_gather(tbl, [r2 + dl])
                acc = c0 * v0 + c1 * v1 + c2 * v2
                plsc.store_scatter(outv, [row, col0 + dl], acc)
            return _

        lax.fori_loop(0, NPAIRS // L, blk_body, None)
        pltpu.sync_copy(
            outv,
            out_hbm.at[pl.ds(chunk * NB, NB), pl.ds(wid * GPT * OUT_DIM,
                                                    GPT * OUT_DIM)])
        return _

    lax.fori_loop(0, B // NB, chunk_body, None)


@jax.jit
def _sc_call(x_t, p_t):
    mesh = plsc.VectorSubcoreMesh(core_axis_name="c", subcore_axis_name="s")
    k = pl.kernel(
        _sc_body,
        mesh=mesh,
        compiler_params=pltpu.CompilerParams(needs_layout_passes=False),
        out_type=jax.ShapeDtypeStruct((B, G * OUT_DIM), jnp.float32),
        scratch_types=[
            pltpu.VMEM((NB * 12,), jnp.float32),
            pltpu.VMEM((TBL,), jnp.float32),
            pltpu.VMEM((NB, GPT * OUT_DIM), jnp.float32),
            pltpu.SemaphoreType.DMA,
        ],
    )
    return k(x_t, p_t)


def kernel(X, params):
    x_t = X.reshape(B, NTILES, GPT * ARITY).transpose(1, 0, 2)
    x_t = x_t.reshape(NTILES, B * GPT * ARITY)
    p_pad = jnp.pad(params, ((0, 0), (0, 0), (0, PAD - OUT_DIM)))
    p_t = p_pad.reshape(NTILES, TBL)
    return _sc_call(x_t, p_t)
